# baseline (speedup 1.0000x reference)
import functools

import jax
import jax.numpy as jnp
from jax import lax
from jax.experimental import pallas as pl
from jax.experimental.pallas import tpu as pltpu
from jax.experimental.pallas import tpu_sc as plsc

_BETA = 0.25
_B, _Q, _K, _C, _T = 8, 64, 512, 256, 2048
_TB = 256
_GRID = _T // _TB

_NC, _NS, _L = 2, 16, 16
_NW = _NC * _NS
_TW = 128
_BG = _B // 2
_NV = _TW // _L
_KC = 128
_NKC = _K // _KC
_KU = 4


def _dense_body(qp_ref, tw_ref, parts_ref, out_ref):
    i = pl.program_id(0)
    qp = qp_ref[...]
    tw = tw_ref[...]

    m = jnp.max(qp, axis=1)
    lse = m + jnp.log(jnp.sum(jnp.exp(qp - m[:, None, :]), axis=1))

    cidx = lax.broadcasted_iota(jnp.int32, (1, _C, 1), 1)
    g = jnp.sum(jnp.where(cidx == tw, qp, 0.0), axis=1)

    part = jnp.sum(g - lse)

    @pl.when(i == 0)
    def _():
        out_ref[0, 0] = (1.0 + _BETA) * jnp.sum(parts_ref[...])

    out_ref[0, 0] += part


def _softmax_loss(qp, tw, parts):
    return pl.pallas_call(
        _dense_body,
        grid=(_GRID,),
        in_specs=[
            pl.BlockSpec((_B, _C, _TB), lambda i: (0, 0, i)),
            pl.BlockSpec((_B, 1, _TB), lambda i: (0, 0, i)),
            pl.BlockSpec((_NW, _L), lambda i: (0, 0)),
        ],
        out_specs=pl.BlockSpec(memory_space=pltpu.SMEM),
        out_shape=jax.ShapeDtypeStruct((1, 1), jnp.float32),
    )(qp, tw, parts)


def _sc_codebook_body(ze_hbm, emb_hbm, out_hbm, ze_v, em_v, s1_v, dm_v, acc_v,
                      sem_z, sem_e0, sem_e1):
    wid = lax.axis_index("s") * _NC + lax.axis_index("c")
    slab = lax.div(wid, 2)
    t0 = slab * _TW
    b0 = lax.rem(wid, 2) * _BG

    cp_z = pltpu.make_async_copy(
        ze_hbm.at[pl.ds(b0, _BG), :, pl.ds(t0, _TW)], ze_v, sem_z)
    esems = [sem_e0, sem_e1]
    cps_e = [
        pltpu.make_async_copy(
            emb_hbm.at[pl.ds(c * _KC, _KC), pl.ds(t0, _TW)],
            em_v.at[c % 2], esems[c % 2])
        for c in range(_NKC)
    ]
    cp_z.start()
    cps_e[0].start()
    cps_e[1].start()
    cp_z.wait()

    acc = jnp.zeros((_L,), jnp.float32)
    for b in range(_BG):
        def s12_step(q, carry):
            s1s, s2s = carry
            s1o, s2o = [], []
            for v in range(_NV):
                x = ze_v[b, q, pl.ds(v * _L, _L)]
                s1o.append(s1s[v] + x)
                s2o.append(s2s[v] + x * x)
            return tuple(s1o), tuple(s2o)

        zer = tuple(jnp.zeros((_L,), jnp.float32) for _ in range(_NV))
        s1s, s2s = lax.fori_loop(0, _Q, s12_step, (zer, zer))
        for v in range(_NV):
            s1_v[b, pl.ds(v * _L, _L)] = s1s[v]
            acc = acc + s2s[v]

    for c in range(_NKC):
        cps_e[c].wait()
        for v in range(_NV):
            s1s = [s1_v[b, pl.ds(v * _L, _L)] for b in range(_BG)]
            init = tuple(jnp.full((_L,), jnp.inf, jnp.float32)
                         for _ in range(_BG))

            @plsc.parallel_loop(0, _KC, step=_KU, unroll=2, carry=init)
            def min_step(k, dmins):
                for u in range(_KU):
                    e = em_v[c % 2, k + u, pl.ds(v * _L, _L)]
                    ne = e * (-2.0)
                    e2q = (e * e) * float(_Q)
                    dmins = tuple(
                        jnp.minimum(dmins[b], e2q + ne * s1s[b])
                        for b in range(_BG)
                    )
                return dmins

            dmins = min_step
            for b in range(_BG):
                if c == 0:
                    dm_v[b, pl.ds(v * _L, _L)] = dmins[b]
                else:
                    dm_v[b, pl.ds(v * _L, _L)] = jnp.minimum(
                        dm_v[b, pl.ds(v * _L, _L)], dmins[b])
        if c + 2 < _NKC:
            cps_e[c + 2].start()

    for v in range(_NV):
        for b in range(_BG):
            acc = acc + dm_v[b, pl.ds(v * _L, _L)]

    acc_v[...] = acc
    pltpu.sync_copy(acc_v, out_hbm.at[wid])


def _codebook_loss(ze, emb):
    mesh = plsc.VectorSubcoreMesh(core_axis_name="c", subcore_axis_name="s")
    fn = functools.partial(
        pl.kernel,
        mesh=mesh,
        out_type=jax.ShapeDtypeStruct((_NW, _L), jnp.float32),
        scratch_types=[
            pltpu.VMEM((_BG, _Q, _TW), jnp.float32),
            pltpu.VMEM((2, _KC, _TW), jnp.float32),
            pltpu.VMEM((_BG, _TW), jnp.float32),
            pltpu.VMEM((_BG, _TW), jnp.float32),
            pltpu.VMEM((_L,), jnp.float32),
            pltpu.SemaphoreType.DMA,
            pltpu.SemaphoreType.DMA,
            pltpu.SemaphoreType.DMA,
        ],
    )(_sc_codebook_body)
    return fn(ze, emb)


def kernel(quant_pred, target_wav, ze, emb):
    tw = target_wav.astype(jnp.int32)
    parts = _codebook_loss(ze, emb)
    total = _softmax_loss(quant_pred, tw, parts)[0, 0]
    return total / float(_B * _T)


# speedup vs baseline: 1.1645x; 1.1645x over previous
import functools

import jax
import jax.numpy as jnp
from jax import lax
from jax.experimental import pallas as pl
from jax.experimental.pallas import tpu as pltpu
from jax.experimental.pallas import tpu_sc as plsc

_BETA = 0.25
_B, _Q, _K, _C, _T = 8, 64, 512, 256, 2048
_TB = 256
_GRID = _T // _TB

_NC, _NS, _L = 2, 16, 16
_NW = _NC * _NS
_TW = 128
_BG = _B // 2
_NV = _TW // _L
_KC = 128
_NKC = _K // _KC
_KU = 4


def _dense_body(qp_ref, tw_ref, out_ref):
    i = pl.program_id(0)
    qp = qp_ref[...]
    tw = tw_ref[...]

    m = jnp.max(qp, axis=1)
    lse = m + jnp.log(jnp.sum(jnp.exp(qp - m[:, None, :]), axis=1))

    cidx = lax.broadcasted_iota(jnp.int32, (1, _C, 1), 1)
    g = jnp.sum(jnp.where(cidx == tw, qp, 0.0), axis=1)

    part = jnp.sum(g - lse)

    @pl.when(i == 0)
    def _():
        out_ref[0, 0] = 0.0

    out_ref[0, 0] += part


def _softmax_loss(qp, tw):
    return pl.pallas_call(
        _dense_body,
        grid=(_GRID,),
        in_specs=[
            pl.BlockSpec((_B, _C, _TB), lambda i: (0, 0, i)),
            pl.BlockSpec((_B, 1, _TB), lambda i: (0, 0, i)),
        ],
        out_specs=pl.BlockSpec(memory_space=pltpu.SMEM),
        out_shape=jax.ShapeDtypeStruct((1, 1), jnp.float32),
    )(qp, tw)


def _sc_codebook_body(ze_hbm, emb_hbm, out_hbm, ze_v, em_v, s1_v, dm_v, acc_v,
                      sem_z, sem_e0, sem_e1):
    wid = lax.axis_index("s") * _NC + lax.axis_index("c")
    slab = lax.div(wid, 2)
    t0 = slab * _TW
    b0 = lax.rem(wid, 2) * _BG

    cp_z = pltpu.make_async_copy(
        ze_hbm.at[pl.ds(b0, _BG), :, pl.ds(t0, _TW)], ze_v, sem_z)
    esems = [sem_e0, sem_e1]
    cps_e = [
        pltpu.make_async_copy(
            emb_hbm.at[pl.ds(c * _KC, _KC), pl.ds(t0, _TW)],
            em_v.at[c % 2], esems[c % 2])
        for c in range(_NKC)
    ]
    cp_z.start()
    cps_e[0].start()
    cps_e[1].start()
    cp_z.wait()

    acc = jnp.zeros((_L,), jnp.float32)
    for b in range(_BG):
        def s12_step(q, carry):
            s1s, s2s = carry
            s1o, s2o = [], []
            for v in range(_NV):
                x = ze_v[b, q, pl.ds(v * _L, _L)]
                s1o.append(s1s[v] + x)
                s2o.append(s2s[v] + x * x)
            return tuple(s1o), tuple(s2o)

        zer = tuple(jnp.zeros((_L,), jnp.float32) for _ in range(_NV))
        s1s, s2s = lax.fori_loop(0, _Q, s12_step, (zer, zer))
        for v in range(_NV):
            s1_v[b, pl.ds(v * _L, _L)] = s1s[v]
            acc = acc + s2s[v]

    for c in range(_NKC):
        cps_e[c].wait()
        for v in range(_NV):
            s1s = [s1_v[b, pl.ds(v * _L, _L)] for b in range(_BG)]
            init = tuple(jnp.full((_L,), jnp.inf, jnp.float32)
                         for _ in range(_BG))

            @plsc.parallel_loop(0, _KC, step=_KU, unroll=2, carry=init)
            def min_step(k, dmins):
                for u in range(_KU):
                    e = em_v[c % 2, k + u, pl.ds(v * _L, _L)]
                    ne = e * (-2.0)
                    e2q = (e * e) * float(_Q)
                    dmins = tuple(
                        jnp.minimum(dmins[b], e2q + ne * s1s[b])
                        for b in range(_BG)
                    )
                return dmins

            dmins = min_step
            for b in range(_BG):
                if c == 0:
                    dm_v[b, pl.ds(v * _L, _L)] = dmins[b]
                else:
                    dm_v[b, pl.ds(v * _L, _L)] = jnp.minimum(
                        dm_v[b, pl.ds(v * _L, _L)], dmins[b])
        if c + 2 < _NKC:
            cps_e[c + 2].start()

    for v in range(_NV):
        for b in range(_BG):
            acc = acc + dm_v[b, pl.ds(v * _L, _L)]

    acc_v[...] = acc
    pltpu.sync_copy(acc_v, out_hbm.at[wid])


def _codebook_loss(ze, emb):
    mesh = plsc.VectorSubcoreMesh(core_axis_name="c", subcore_axis_name="s")
    fn = functools.partial(
        pl.kernel,
        mesh=mesh,
        out_type=jax.ShapeDtypeStruct((_NW, _L), jnp.float32),
        scratch_types=[
            pltpu.VMEM((_BG, _Q, _TW), jnp.float32),
            pltpu.VMEM((2, _KC, _TW), jnp.float32),
            pltpu.VMEM((_BG, _TW), jnp.float32),
            pltpu.VMEM((_BG, _TW), jnp.float32),
            pltpu.VMEM((_L,), jnp.float32),
            pltpu.SemaphoreType.DMA,
            pltpu.SemaphoreType.DMA,
            pltpu.SemaphoreType.DMA,
        ],
    )(_sc_codebook_body)
    return fn(ze, emb)


def kernel(quant_pred, target_wav, ze, emb):
    tw = target_wav.astype(jnp.int32)
    parts = _codebook_loss(ze, emb)
    soft = _softmax_loss(quant_pred, tw)[0, 0]
    total = soft + (1.0 + _BETA) * jnp.sum(parts)
    return total / float(_B * _T)
